# Initial kernel scaffold; baseline (speedup 1.0000x reference)
import jax
import jax.numpy as jnp
from jax.experimental import pallas as pl


def kernel(X, vertex, edges, V_class, E_class, W, b, att_v, att_e):
    raise NotImplementedError("write your pallas kernel here")


# speedup vs baseline: 1.0167x; 1.0167x over previous
import functools

import jax
import jax.numpy as jnp
from jax import lax
from jax.experimental import pallas as pl

N = 10000
NNZ = 160000
EDGE_NUM = 20000
HEADS = 8
OUT_CH = 64
NEG_SLOPE = 0.2


def _mm_kernel(x_ref, w_ref, o_ref):
    o_ref[...] = lax.dot_general(
        x_ref[...], w_ref[...], (((1,), (1,)), ((), ())),
        preferred_element_type=jnp.float32)


def _project(X, W):
    n, cin = X.shape
    cout = W.shape[0]
    blk = 1000
    return pl.pallas_call(
        _mm_kernel,
        grid=(n // blk,),
        in_specs=[
            pl.BlockSpec((blk, cin), lambda i: (i, 0)),
            pl.BlockSpec((cout, cin), lambda i: (0, 0)),
        ],
        out_specs=pl.BlockSpec((blk, cout), lambda i: (i, 0)),
        out_shape=jax.ShapeDtypeStruct((n, cout), jnp.float32),
    )(X, W)


def kernel(X, vertex, edges, V_class, E_class, W, b, att_v, att_e):
    X0 = _project(X, W) + b
    Xh = X0.reshape(N, HEADS, OUT_CH)

    P = jnp.einsum('nhk,chk->nch', Xh, att_e)
    w_nnz = jnp.exp(jax.nn.leaky_relu(P[vertex, E_class, :], NEG_SLOPE))

    num_e = jnp.zeros((EDGE_NUM, HEADS, OUT_CH), jnp.float32).at[edges].add(
        Xh[vertex] * w_nnz[..., None])
    den_e = jnp.zeros((EDGE_NUM, HEADS), jnp.float32).at[edges].add(w_nnz)
    Xe = num_e / (den_e[..., None] + 1e-16)

    Q = jnp.einsum('ehk,chk->ech', Xe, att_v)
    u_nnz = jnp.exp(jax.nn.leaky_relu(Q[edges, V_class, :], NEG_SLOPE))

    num_v = jnp.zeros((N, HEADS, OUT_CH), jnp.float32).at[vertex].add(
        Xe[edges] * u_nnz[..., None])
    den_v = jnp.zeros((N, HEADS), jnp.float32).at[vertex].add(u_nnz)
    Xv = num_v / (den_v[..., None] + 1e-16)
    return Xv.reshape(N, HEADS * OUT_CH)


# speedup vs baseline: 11.2796x; 11.0948x over previous
import functools

import jax
import jax.numpy as jnp
from jax import lax
from jax.experimental import pallas as pl
from jax.experimental.pallas import tpu as pltpu
from jax.experimental.pallas import tpu_sc as plsc

N = 10000
NNZ = 160000
EDGE_NUM = 20000
IN_CH = 256
HEADS = 8
OUT_CH = 64
D = HEADS * OUT_CH
NEG_SLOPE = 0.2

NT = 32
SUBS = 5
MB = 64
NNZ_PAD = NNZ + 2 * MB
RP_LEN = 176

N_PADB = 10240
EDGE_PAD = 20480
SUB_A = EDGE_PAD // (NT * SUBS)
SUB_B = N_PADB // (NT * SUBS)


def _proj_kernel(x_ref, w_ref, b_ref, be_ref, x0_ref, p_ref):
    x0 = lax.dot_general(x_ref[...], w_ref[...], (((1,), (1,)), ((), ())),
                         preferred_element_type=jnp.float32) + b_ref[...]
    x0_ref[...] = x0
    p_ref[...] = jnp.dot(x0, be_ref[...], preferred_element_type=jnp.float32)


def _project(X, W, b2, Be64):
    blk = 1000
    return pl.pallas_call(
        _proj_kernel,
        grid=(N // blk,),
        in_specs=[
            pl.BlockSpec((blk, IN_CH), lambda i: (i, 0)),
            pl.BlockSpec((D, IN_CH), lambda i: (0, 0)),
            pl.BlockSpec((1, D), lambda i: (0, 0)),
            pl.BlockSpec((D, 128), lambda i: (0, 0)),
        ],
        out_specs=[
            pl.BlockSpec((blk, D), lambda i: (i, 0)),
            pl.BlockSpec((blk, 128), lambda i: (i, 0)),
        ],
        out_shape=[
            jax.ShapeDtypeStruct((N, D), jnp.float32),
            jax.ShapeDtypeStruct((N, 128), jnp.float32),
        ],
    )(X, W, b2, Be64)


def _score_kernel(xe_ref, bv_ref, q_ref):
    q_ref[...] = jnp.dot(xe_ref[...], bv_ref[...],
                         preferred_element_type=jnp.float32)


def _score(Xe, Bv64):
    blk = 2048
    return pl.pallas_call(
        _score_kernel,
        grid=(EDGE_PAD // blk,),
        in_specs=[
            pl.BlockSpec((blk, D), lambda i: (i, 0)),
            pl.BlockSpec((D, 128), lambda i: (0, 0)),
        ],
        out_specs=pl.BlockSpec((blk, 128), lambda i: (i, 0)),
        out_shape=jax.ShapeDtypeStruct((EDGE_PAD, 128), jnp.float32),
    )(Xe, Bv64)


def _make_agg(n_rows_out, sub):
    mesh = plsc.VectorSubcoreMesh(core_axis_name="c", subcore_axis_name="s")

    @functools.partial(
        pl.kernel, mesh=mesh,
        out_type=jax.ShapeDtypeStruct((n_rows_out, D), jnp.float32),
        scratch_types=[
            pltpu.VMEM((MB,), jnp.int32),
            pltpu.VMEM((MB + 16,), jnp.int32),
            pltpu.VMEM((MB + 16,), jnp.int32),
            pltpu.VMEM((MB, D), jnp.float32),
            pltpu.VMEM((MB, 128), jnp.float32),
            pltpu.VMEM((sub, D), jnp.float32),
            pltpu.VMEM((sub, 16), jnp.float32),
            pltpu.VMEM((RP_LEN,), jnp.int32),
            pltpu.SemaphoreType.DMA,
            pltpu.SemaphoreType.DMA,
        ])
    def agg(table_hbm, p_hbm, src_hbm, cls_hbm, dst_hbm, rp_hbm, out_hbm,
            idx_v, cls_v, dst_v, rows_v, sc_v, stag_v, den_v, rp_v, sem0, sem1):
        wid = lax.axis_index("s") * 2 + lax.axis_index("c")
        pltpu.sync_copy(rp_hbm, rp_v)
        zv = jnp.zeros((16,), jnp.float32)

        def sub_body(s, _):
            gsub = wid * SUBS + s
            base_row = pl.multiple_of(gsub * sub, 8)
            rp_pair = rp_v[pl.ds(gsub, 16)]
            n_lo = rp_pair[0]
            n_hi = rp_pair[1]
            nb0 = jnp.bitwise_and(n_lo, jnp.int32(~7))
            nblk = (n_hi - nb0 + (MB - 1)) // MB

            def zero_row(e, _):
                for k in range(D // 16):
                    stag_v[e, pl.ds(k * 16, 16)] = zv
                den_v[e, :] = zv
                return 0
            lax.fori_loop(0, sub, zero_row, 0)

            def blk(j, _):
                nb = pl.multiple_of(nb0 + j * MB, 8)
                pltpu.sync_copy(src_hbm.at[pl.ds(nb, MB)], idx_v)
                pltpu.sync_copy(cls_hbm.at[pl.ds(nb, MB)],
                                cls_v.at[pl.ds(0, MB)])
                pltpu.sync_copy(dst_hbm.at[pl.ds(nb, MB)],
                                dst_v.at[pl.ds(0, MB)])
                pltpu.async_copy(p_hbm.at[idx_v], sc_v, sem0).wait()
                pltpu.async_copy(table_hbm.at[idx_v], rows_v, sem1).wait()

                def member(i, _):
                    g = nb + i
                    valid = jnp.logical_and(g >= n_lo, g < n_hi)

                    @pl.when(valid)
                    def _():
                        dloc = dst_v[pl.ds(i, 16)][0] - base_row
                        cls = cls_v[pl.ds(i, 16)][0]
                        srow = sc_v[i, pl.ds(cls * 16, 16)]
                        wrow = jnp.exp(jnp.maximum(srow, srow * NEG_SLOPE))
                        for h in range(HEADS):
                            ws = wrow[h]
                            for q in range(OUT_CH // 16):
                                c = h * OUT_CH + q * 16
                                stag_v[dloc, pl.ds(c, 16)] = (
                                    stag_v[dloc, pl.ds(c, 16)]
                                    + rows_v[i, pl.ds(c, 16)] * ws)
                        den_v[dloc, :] = den_v[dloc, :] + wrow
                    return 0
                lax.fori_loop(0, MB, member, 0)
                return 0
            lax.fori_loop(0, nblk, blk, 0)

            def recip_row(e, _):
                den_v[e, :] = 1.0 / (den_v[e, :] + 1e-16)
                return 0
            lax.fori_loop(0, sub, recip_row, 0)

            def scale_row(e, _):
                drow = den_v[e, :]
                for h in range(HEADS):
                    r = drow[h]
                    for q in range(OUT_CH // 16):
                        c = h * OUT_CH + q * 16
                        stag_v[e, pl.ds(c, 16)] = stag_v[e, pl.ds(c, 16)] * r
                return 0
            lax.fori_loop(0, sub, scale_row, 0)

            pltpu.sync_copy(stag_v, out_hbm.at[pl.ds(base_row, sub)])
            return 0
        lax.fori_loop(0, SUBS, sub_body, 0)

    return agg


_agg_a = _make_agg(EDGE_PAD, SUB_A)
_agg_b = _make_agg(N_PADB, SUB_B)


def _att_mat(att):
    m = jnp.einsum('chk,hg->hkcg', att, jnp.eye(HEADS, dtype=att.dtype))
    m = jnp.pad(m, ((0, 0), (0, 0), (0, 4), (0, 16 - HEADS)))
    return m.reshape(D, 128)


def _csr(dst_ids, src_ids, cls_ids, n_rows, sub):
    order = jnp.argsort(dst_ids)
    dst_s = dst_ids[order].astype(jnp.int32)
    src_s = src_ids[order].astype(jnp.int32)
    gid_s = cls_ids[order].astype(jnp.int32)
    bases = jnp.arange(0, n_rows + 1, sub, dtype=jnp.int32)
    rp = jnp.searchsorted(dst_s, bases).astype(jnp.int32)
    rp = jnp.pad(rp, (0, RP_LEN - rp.shape[0]), constant_values=NNZ)
    pad = NNZ_PAD - NNZ
    dst_s = jnp.pad(dst_s, (0, pad))
    src_s = jnp.pad(src_s, (0, pad))
    gid_s = jnp.pad(gid_s, (0, pad))
    return dst_s, src_s, gid_s, rp


def kernel(X, vertex, edges, V_class, E_class, W, b, att_v, att_e):
    Be64 = _att_mat(att_e)
    Bv64 = _att_mat(att_v)
    X0, P = _project(X, W, b.reshape(1, D), Be64)

    dstA, srcA, clsA, rpA = _csr(edges, vertex, E_class, EDGE_PAD, SUB_A)
    Xe = _agg_a(X0, P, srcA, clsA, dstA, rpA)

    Q = _score(Xe, Bv64)

    dstB, srcB, clsB, rpB = _csr(vertex, edges, V_class, N_PADB, SUB_B)
    Xv = _agg_b(Xe, Q, srcB, clsB, dstB, rpB)
    return Xv[:N]


# speedup vs baseline: 12.9263x; 1.1460x over previous
import functools

import jax
import jax.numpy as jnp
from jax import lax
from jax.experimental import pallas as pl
from jax.experimental.pallas import tpu as pltpu
from jax.experimental.pallas import tpu_sc as plsc

N = 10000
NNZ = 160000
EDGE_NUM = 20000
IN_CH = 256
HEADS = 8
OUT_CH = 64
D = HEADS * OUT_CH
NEG_SLOPE = 0.2

NT = 32
MB = 64
SUB = 64
NNZ_PAD = NNZ + 2 * MB
RP_LEN = 344

N_PADB = 10240
EDGE_PAD = 20480
SUBS_A = EDGE_PAD // (NT * SUB)
SUBS_B = N_PADB // (NT * SUB)


def _proj_kernel(x_ref, w_ref, b_ref, be_ref, x0_ref, p_ref):
    x0 = lax.dot_general(x_ref[...], w_ref[...], (((1,), (1,)), ((), ())),
                         preferred_element_type=jnp.float32) + b_ref[...]
    x0_ref[...] = x0
    p_ref[...] = jnp.dot(x0, be_ref[...], preferred_element_type=jnp.float32)


def _project(X, W, b2, Be64):
    blk = 1000
    return pl.pallas_call(
        _proj_kernel,
        grid=(N // blk,),
        in_specs=[
            pl.BlockSpec((blk, IN_CH), lambda i: (i, 0)),
            pl.BlockSpec((D, IN_CH), lambda i: (0, 0)),
            pl.BlockSpec((1, D), lambda i: (0, 0)),
            pl.BlockSpec((D, 128), lambda i: (0, 0)),
        ],
        out_specs=[
            pl.BlockSpec((blk, D), lambda i: (i, 0)),
            pl.BlockSpec((blk, 128), lambda i: (i, 0)),
        ],
        out_shape=[
            jax.ShapeDtypeStruct((N, D), jnp.float32),
            jax.ShapeDtypeStruct((N, 128), jnp.float32),
        ],
    )(X, W, b2, Be64)


def _score_kernel(xe_ref, bv_ref, q_ref):
    q_ref[...] = jnp.dot(xe_ref[...], bv_ref[...],
                         preferred_element_type=jnp.float32)


def _score(Xe, Bv64):
    blk = 2048
    return pl.pallas_call(
        _score_kernel,
        grid=(EDGE_PAD // blk,),
        in_specs=[
            pl.BlockSpec((blk, D), lambda i: (i, 0)),
            pl.BlockSpec((D, 128), lambda i: (0, 0)),
        ],
        out_specs=pl.BlockSpec((blk, 128), lambda i: (i, 0)),
        out_shape=jax.ShapeDtypeStruct((EDGE_PAD, 128), jnp.float32),
    )(Xe, Bv64)


def _make_agg(n_rows_out, subs):
    mesh = plsc.VectorSubcoreMesh(core_axis_name="c", subcore_axis_name="s")

    @functools.partial(
        pl.kernel, mesh=mesh,
        out_type=jax.ShapeDtypeStruct((n_rows_out, D), jnp.float32),
        scratch_types=[
            pltpu.VMEM((MB,), jnp.int32),
            pltpu.VMEM((MB,), jnp.int32),
            pltpu.VMEM((MB + 16,), jnp.int32),
            pltpu.VMEM((MB + 16,), jnp.int32),
            pltpu.VMEM((MB, D), jnp.float32),
            pltpu.VMEM((MB, D), jnp.float32),
            pltpu.VMEM((MB, 128), jnp.float32),
            pltpu.VMEM((MB, 128), jnp.float32),
            pltpu.VMEM((SUB, D), jnp.float32),
            pltpu.VMEM((SUB, 16), jnp.float32),
            pltpu.VMEM((RP_LEN,), jnp.int32),
            pltpu.SemaphoreType.DMA,
            pltpu.SemaphoreType.DMA,
            pltpu.SemaphoreType.DMA,
            pltpu.SemaphoreType.DMA,
            pltpu.SemaphoreType.DMA,
            pltpu.SemaphoreType.DMA,
        ])
    def agg(table_hbm, p_hbm, src_hbm, desc_hbm, rp_hbm, out_hbm,
            idx0, idx1, dsc0, dsc1, rows0, rows1, sc0, sc1,
            stag_v, den_v, rp_v, s1_0, s1_1, sr0, sr1, ss0, ss1):
        idxb = (idx0, idx1)
        dscb = (dsc0, dsc1)
        rowsb = (rows0, rows1)
        scb = (sc0, sc1)
        s1 = (s1_0, s1_1)
        sr = (sr0, sr1)
        ss = (ss0, ss1)
        wid = lax.axis_index("s") * 2 + lax.axis_index("c")
        pltpu.sync_copy(rp_hbm, rp_v)
        zv = jnp.zeros((16,), jnp.float32)

        def sub_body(s, _):
            gsub = wid * subs + s
            base_row = pl.multiple_of(gsub * SUB, 8)
            rp_pair = rp_v[pl.ds(gsub, 16)]
            n_lo = rp_pair[0]
            n_hi = rp_pair[1]
            nb0 = jnp.bitwise_and(n_lo, jnp.int32(~7))
            nblk = (n_hi - nb0 + (MB - 1)) // MB

            def zero_row(e, _):
                for k in range(D // 16):
                    stag_v[e, pl.ds(k * 16, 16)] = zv
                den_v[e, :] = zv
                return 0
            lax.fori_loop(0, SUB, zero_row, 0)

            def issue_s1(j, b):
                nb = pl.multiple_of(nb0 + j * MB, 8)
                pltpu.async_copy(src_hbm.at[pl.ds(nb, MB)], idxb[b], s1[b])
                pltpu.async_copy(desc_hbm.at[pl.ds(nb, MB)],
                                 dscb[b].at[pl.ds(0, MB)], s1[b])

            def wait_s1(b):
                pltpu.make_async_copy(src_hbm.at[pl.ds(0, MB)],
                                      idxb[b], s1[b]).wait()
                pltpu.make_async_copy(desc_hbm.at[pl.ds(0, MB)],
                                      dscb[b].at[pl.ds(0, MB)], s1[b]).wait()

            def issue_s2(b):
                pltpu.async_copy(p_hbm.at[idxb[b]], scb[b], ss[b])
                pltpu.async_copy(table_hbm.at[idxb[b]], rowsb[b], sr[b])

            def wait_s2(b):
                pltpu.make_async_copy(p_hbm.at[idxb[b]], scb[b], ss[b]).wait()
                pltpu.make_async_copy(table_hbm.at[idxb[b]],
                                      rowsb[b], sr[b]).wait()

            def process(j, b):
                nb = nb0 + j * MB
                rows_r = rowsb[b]
                sc_r = scb[b]
                dsc_r = dscb[b]

                def member(i, _):
                    g = nb + i
                    valid = jnp.logical_and(g >= n_lo, g < n_hi)
                    mf = jnp.where(valid, 1.0, 0.0)
                    dsc = dsc_r[pl.ds(i, 16)][0]
                    dloc = lax.shift_right_logical(dsc, 2)
                    c16 = lax.shift_left(jnp.bitwise_and(dsc, 3), 4)
                    srow = sc_r[i, pl.ds(c16, 16)]
                    wrow = jnp.exp(jnp.maximum(srow, srow * NEG_SLOPE)) * mf
                    for h in range(HEADS):
                        ws = wrow[h]
                        for q in range(OUT_CH // 16):
                            c = h * OUT_CH + q * 16
                            stag_v[dloc, pl.ds(c, 16)] = (
                                stag_v[dloc, pl.ds(c, 16)]
                                + rows_r[i, pl.ds(c, 16)] * ws)
                    den_v[dloc, :] = den_v[dloc, :] + wrow
                    return 0
                lax.fori_loop(0, MB, member, 0)

            @pl.when(nblk > 0)
            def _():
                issue_s1(0, 0)

                @pl.when(nblk > 1)
                def _():
                    issue_s1(1, 1)
                wait_s1(0)
                issue_s2(0)

                def pair(jj, _):
                    for b in (0, 1):
                        j = 2 * jj + b

                        @pl.when(j < nblk)
                        def _():
                            wait_s2(b)

                            @pl.when(j + 1 < nblk)
                            def _():
                                wait_s1(1 - b)
                                issue_s2(1 - b)
                            process(j, b)

                            @pl.when(j + 2 < nblk)
                            def _():
                                issue_s1(j + 2, b)
                    return 0
                lax.fori_loop(0, (nblk + 1) // 2, pair, 0)

            def scale_row(e, _):
                rec = 1.0 / (den_v[e, :] + 1e-16)
                for h in range(HEADS):
                    r = rec[h]
                    for q in range(OUT_CH // 16):
                        c = h * OUT_CH + q * 16
                        stag_v[e, pl.ds(c, 16)] = stag_v[e, pl.ds(c, 16)] * r
                return 0
            lax.fori_loop(0, SUB, scale_row, 0)

            pltpu.sync_copy(stag_v, out_hbm.at[pl.ds(base_row, SUB)])
            return 0
        lax.fori_loop(0, subs, sub_body, 0)

    return agg


_agg_a = _make_agg(EDGE_PAD, SUBS_A)
_agg_b = _make_agg(N_PADB, SUBS_B)


def _att_mat(att):
    m = jnp.einsum('chk,hg->hkcg', att, jnp.eye(HEADS, dtype=att.dtype))
    m = jnp.pad(m, ((0, 0), (0, 0), (0, 4), (0, 16 - HEADS)))
    return m.reshape(D, 128)


def _csr(dst_ids, src_ids, cls_ids, n_rows):
    order = jnp.argsort(dst_ids)
    dst_s = dst_ids[order].astype(jnp.int32)
    src_s = src_ids[order].astype(jnp.int32)
    cls_s = cls_ids[order].astype(jnp.int32)
    desc_s = jnp.bitwise_and(dst_s, SUB - 1) * 4 + cls_s
    bases = jnp.arange(0, n_rows + 1, SUB, dtype=jnp.int32)
    rp = jnp.searchsorted(dst_s, bases).astype(jnp.int32)
    rp = jnp.pad(rp, (0, RP_LEN - rp.shape[0]), constant_values=NNZ)
    pad = NNZ_PAD - NNZ
    src_s = jnp.pad(src_s, (0, pad))
    desc_s = jnp.pad(desc_s, (0, pad))
    return src_s, desc_s, rp


def kernel(X, vertex, edges, V_class, E_class, W, b, att_v, att_e):
    Be64 = _att_mat(att_e)
    Bv64 = _att_mat(att_v)
    X0, P = _project(X, W, b.reshape(1, D), Be64)

    srcA, descA, rpA = _csr(edges, vertex, E_class, EDGE_PAD)
    Xe = _agg_a(X0, P, srcA, descA, rpA)

    Q = _score(Xe, Bv64)

    srcB, descB, rpB = _csr(vertex, edges, V_class, N_PADB)
    Xv = _agg_b(Xe, Q, srcB, descB, rpB)
    return Xv[:N]


# speedup vs baseline: 14.9375x; 1.1556x over previous
import functools

import jax
import jax.numpy as jnp
from jax import lax
from jax.experimental import pallas as pl
from jax.experimental.pallas import tpu as pltpu
from jax.experimental.pallas import tpu_sc as plsc

N = 10000
NNZ = 160000
EDGE_NUM = 20000
IN_CH = 256
HEADS = 8
OUT_CH = 64
D = HEADS * OUT_CH
NEG_SLOPE = 0.2

NT = 32
MB = 64
SUB = 64
NNZ_PAD = NNZ + 2 * MB
RP_LEN = 344

N_PADB = 10240
EDGE_PAD = 20480
SUBS_A = EDGE_PAD // (NT * SUB)
SUBS_B = N_PADB // (NT * SUB)


def _proj_kernel(x_ref, w_ref, b_ref, be_ref, x0_ref, p_ref):
    x0 = lax.dot_general(x_ref[...], w_ref[...], (((1,), (1,)), ((), ())),
                         preferred_element_type=jnp.float32) + b_ref[...]
    x0_ref[...] = x0
    p_ref[...] = jnp.dot(x0, be_ref[...], preferred_element_type=jnp.float32)


def _project(X, W, b2, Be64):
    blk = 1000
    return pl.pallas_call(
        _proj_kernel,
        grid=(N // blk,),
        in_specs=[
            pl.BlockSpec((blk, IN_CH), lambda i: (i, 0)),
            pl.BlockSpec((D, IN_CH), lambda i: (0, 0)),
            pl.BlockSpec((1, D), lambda i: (0, 0)),
            pl.BlockSpec((D, 128), lambda i: (0, 0)),
        ],
        out_specs=[
            pl.BlockSpec((blk, D), lambda i: (i, 0)),
            pl.BlockSpec((blk, 128), lambda i: (i, 0)),
        ],
        out_shape=[
            jax.ShapeDtypeStruct((N, D), jnp.float32),
            jax.ShapeDtypeStruct((N, 128), jnp.float32),
        ],
    )(X, W, b2, Be64)


def _score_kernel(xe_ref, bv_ref, q_ref):
    q_ref[...] = jnp.dot(xe_ref[...], bv_ref[...],
                         preferred_element_type=jnp.float32)


def _score(Xe, Bv64):
    blk = 2048
    return pl.pallas_call(
        _score_kernel,
        grid=(EDGE_PAD // blk,),
        in_specs=[
            pl.BlockSpec((blk, D), lambda i: (i, 0)),
            pl.BlockSpec((D, 128), lambda i: (0, 0)),
        ],
        out_specs=pl.BlockSpec((blk, 128), lambda i: (i, 0)),
        out_shape=jax.ShapeDtypeStruct((EDGE_PAD, 128), jnp.float32),
    )(Xe, Bv64)


def _make_agg(n_rows_out, subs):
    mesh = plsc.VectorSubcoreMesh(core_axis_name="c", subcore_axis_name="s")

    @functools.partial(
        pl.kernel, mesh=mesh,
        out_type=jax.ShapeDtypeStruct((n_rows_out, D), jnp.float32),
        scratch_types=[
            pltpu.VMEM((MB,), jnp.int32),
            pltpu.VMEM((MB,), jnp.int32),
            pltpu.VMEM((MB + 16,), jnp.int32),
            pltpu.VMEM((MB + 16,), jnp.int32),
            pltpu.VMEM((MB, D), jnp.float32),
            pltpu.VMEM((MB, D), jnp.float32),
            pltpu.VMEM((MB, 128), jnp.float32),
            pltpu.VMEM((MB, 128), jnp.float32),
            pltpu.VMEM((SUB, D), jnp.float32),
            pltpu.VMEM((SUB, 16), jnp.float32),
            pltpu.VMEM((RP_LEN,), jnp.int32),
            pltpu.SemaphoreType.DMA,
            pltpu.SemaphoreType.DMA,
            pltpu.SemaphoreType.DMA,
            pltpu.SemaphoreType.DMA,
            pltpu.SemaphoreType.DMA,
            pltpu.SemaphoreType.DMA,
        ])
    def agg(table_hbm, p_hbm, src_hbm, desc_hbm, rp_hbm, out_hbm,
            idx0, idx1, dsc0, dsc1, rows0, rows1, sc0, sc1,
            stag_v, den_v, rp_v, s1_0, s1_1, sr0, sr1, ss0, ss1):
        idxb = (idx0, idx1)
        dscb = (dsc0, dsc1)
        rowsb = (rows0, rows1)
        scb = (sc0, sc1)
        s1 = (s1_0, s1_1)
        sr = (sr0, sr1)
        ss = (ss0, ss1)
        wid = lax.axis_index("s") * 2 + lax.axis_index("c")
        pltpu.sync_copy(rp_hbm, rp_v)
        zv = jnp.zeros((16,), jnp.float32)

        def sub_body(s, _):
            gsub = wid * subs + s
            base_row = pl.multiple_of(gsub * SUB, 8)
            rp_pair = rp_v[pl.ds(gsub, 16)]
            n_lo = rp_pair[0]
            n_hi = rp_pair[1]
            nb0 = jnp.bitwise_and(n_lo, jnp.int32(~7))
            nblk = (n_hi - nb0 + (MB - 1)) // MB

            def zero_row(e, _):
                for k in range(D // 16):
                    stag_v[e, pl.ds(k * 16, 16)] = zv
                den_v[e, :] = zv
                return 0
            lax.fori_loop(0, SUB, zero_row, 0)

            def issue_s1(j, b):
                nb = pl.multiple_of(nb0 + j * MB, 8)
                pltpu.async_copy(src_hbm.at[pl.ds(nb, MB)], idxb[b], s1[b])
                pltpu.async_copy(desc_hbm.at[pl.ds(nb, MB)],
                                 dscb[b].at[pl.ds(0, MB)], s1[b])

            def wait_s1(b):
                pltpu.make_async_copy(src_hbm.at[pl.ds(0, MB)],
                                      idxb[b], s1[b]).wait()
                pltpu.make_async_copy(desc_hbm.at[pl.ds(0, MB)],
                                      dscb[b].at[pl.ds(0, MB)], s1[b]).wait()

            def issue_s2(b):
                pltpu.async_copy(p_hbm.at[idxb[b]], scb[b], ss[b])
                pltpu.async_copy(table_hbm.at[idxb[b]], rowsb[b], sr[b])

            def wait_s2(b):
                pltpu.make_async_copy(p_hbm.at[idxb[b]], scb[b], ss[b]).wait()
                pltpu.make_async_copy(table_hbm.at[idxb[b]],
                                      rowsb[b], sr[b]).wait()

            def process(j, b):
                nb = nb0 + j * MB
                rows_r = rowsb[b]
                sc_r = scb[b]
                dsc_r = dscb[b]

                def member(i, _):
                    g = nb + i
                    valid = jnp.logical_and(g >= n_lo, g < n_hi)
                    mf = jnp.where(valid, 1.0, 0.0)
                    dsc = dsc_r[pl.ds(i, 16)][0]
                    dloc = lax.shift_right_logical(dsc, 2)
                    c16 = lax.shift_left(jnp.bitwise_and(dsc, 3), 4)
                    srow = sc_r[i, pl.ds(c16, 16)]
                    wrow = jnp.exp(jnp.maximum(srow, srow * NEG_SLOPE)) * mf
                    for h in range(HEADS):
                        ws = wrow[h]
                        for q in range(OUT_CH // 16):
                            c = h * OUT_CH + q * 16
                            plsc.addupdate(stag_v.at[dloc, pl.ds(c, 16)],
                                           rows_r[i, pl.ds(c, 16)] * ws)
                    plsc.addupdate(den_v.at[dloc, :], wrow)
                    return 0
                lax.fori_loop(0, MB, member, 0, unroll=4)

            @pl.when(nblk > 0)
            def _():
                issue_s1(0, 0)

                @pl.when(nblk > 1)
                def _():
                    issue_s1(1, 1)
                wait_s1(0)
                issue_s2(0)

                def pair(jj, _):
                    for b in (0, 1):
                        j = 2 * jj + b

                        @pl.when(j < nblk)
                        def _():
                            wait_s2(b)

                            @pl.when(j + 1 < nblk)
                            def _():
                                wait_s1(1 - b)
                                issue_s2(1 - b)
                            process(j, b)

                            @pl.when(j + 2 < nblk)
                            def _():
                                issue_s1(j + 2, b)
                    return 0
                lax.fori_loop(0, (nblk + 1) // 2, pair, 0)

            def scale_row(e, _):
                rec = 1.0 / (den_v[e, :] + 1e-16)
                for h in range(HEADS):
                    r = rec[h]
                    for q in range(OUT_CH // 16):
                        c = h * OUT_CH + q * 16
                        stag_v[e, pl.ds(c, 16)] = stag_v[e, pl.ds(c, 16)] * r
                return 0
            lax.fori_loop(0, SUB, scale_row, 0)

            pltpu.sync_copy(stag_v, out_hbm.at[pl.ds(base_row, SUB)])
            return 0
        lax.fori_loop(0, subs, sub_body, 0)

    return agg


_agg_a = _make_agg(EDGE_PAD, SUBS_A)
_agg_b = _make_agg(N_PADB, SUBS_B)


def _att_mat(att):
    m = jnp.einsum('chk,hg->hkcg', att, jnp.eye(HEADS, dtype=att.dtype))
    m = jnp.pad(m, ((0, 0), (0, 0), (0, 4), (0, 16 - HEADS)))
    return m.reshape(D, 128)


def _csr(dst_ids, src_ids, cls_ids, n_rows):
    order = jnp.argsort(dst_ids)
    dst_s = dst_ids[order].astype(jnp.int32)
    src_s = src_ids[order].astype(jnp.int32)
    cls_s = cls_ids[order].astype(jnp.int32)
    desc_s = jnp.bitwise_and(dst_s, SUB - 1) * 4 + cls_s
    bases = jnp.arange(0, n_rows + 1, SUB, dtype=jnp.int32)
    rp = jnp.searchsorted(dst_s, bases).astype(jnp.int32)
    rp = jnp.pad(rp, (0, RP_LEN - rp.shape[0]), constant_values=NNZ)
    pad = NNZ_PAD - NNZ
    src_s = jnp.pad(src_s, (0, pad))
    desc_s = jnp.pad(desc_s, (0, pad))
    return src_s, desc_s, rp


def kernel(X, vertex, edges, V_class, E_class, W, b, att_v, att_e):
    Be64 = _att_mat(att_e)
    Bv64 = _att_mat(att_v)
    X0, P = _project(X, W, b.reshape(1, D), Be64)

    srcA, descA, rpA = _csr(edges, vertex, E_class, EDGE_PAD)
    Xe = _agg_a(X0, P, srcA, descA, rpA)

    Q = _score(Xe, Bv64)

    srcB, descB, rpB = _csr(vertex, edges, V_class, N_PADB)
    Xv = _agg_b(Xe, Q, srcB, descB, rpB)
    return Xv[:N]


# speedup vs baseline: 28.1188x; 1.8824x over previous
import functools

import jax
import jax.numpy as jnp
from jax import lax
from jax.experimental import pallas as pl
from jax.experimental.pallas import tpu as pltpu
from jax.experimental.pallas import tpu_sc as plsc

N = 10000
NNZ = 160000
EDGE_NUM = 20000
IN_CH = 256
HEADS = 8
OUT_CH = 64
D = HEADS * OUT_CH
NEG_SLOPE = 0.2

NT = 32
MB = 64
SUB = 64
NNZ_PAD = NNZ + 2 * MB
RP_LEN = 344

N_PADB = 10240
EDGE_PAD = 20480
SUBS_A = EDGE_PAD // (NT * SUB)
SUBS_B = N_PADB // (NT * SUB)


def _proj_kernel(x_ref, w_ref, b_ref, be_ref, x0_ref, p_ref):
    x0 = lax.dot_general(x_ref[...], w_ref[...], (((1,), (1,)), ((), ())),
                         preferred_element_type=jnp.float32) + b_ref[...]
    x0_ref[...] = x0
    p_ref[...] = jnp.dot(x0, be_ref[...], preferred_element_type=jnp.float32)


def _project(X, W, b2, Be64):
    blk = 1000
    return pl.pallas_call(
        _proj_kernel,
        grid=(N // blk,),
        in_specs=[
            pl.BlockSpec((blk, IN_CH), lambda i: (i, 0)),
            pl.BlockSpec((D, IN_CH), lambda i: (0, 0)),
            pl.BlockSpec((1, D), lambda i: (0, 0)),
            pl.BlockSpec((D, 128), lambda i: (0, 0)),
        ],
        out_specs=[
            pl.BlockSpec((blk, D), lambda i: (i, 0)),
            pl.BlockSpec((blk, 128), lambda i: (i, 0)),
        ],
        out_shape=[
            jax.ShapeDtypeStruct((N, D), jnp.float32),
            jax.ShapeDtypeStruct((N, 128), jnp.float32),
        ],
    )(X, W, b2, Be64)


def _score_kernel(xe_ref, bv_ref, q_ref):
    q_ref[...] = jnp.dot(xe_ref[...], bv_ref[...],
                         preferred_element_type=jnp.float32)


def _score(Xe, Bv64):
    blk = 2048
    return pl.pallas_call(
        _score_kernel,
        grid=(EDGE_PAD // blk,),
        in_specs=[
            pl.BlockSpec((blk, D), lambda i: (i, 0)),
            pl.BlockSpec((D, 128), lambda i: (0, 0)),
        ],
        out_specs=pl.BlockSpec((blk, 128), lambda i: (i, 0)),
        out_shape=jax.ShapeDtypeStruct((EDGE_PAD, 128), jnp.float32),
    )(Xe, Bv64)


def _make_agg(n_rows_out, subs):
    mesh = plsc.VectorSubcoreMesh(core_axis_name="c", subcore_axis_name="s")

    @functools.partial(
        pl.kernel, mesh=mesh,
        out_type=jax.ShapeDtypeStruct((n_rows_out, D), jnp.float32),
        scratch_types=[
            pltpu.VMEM((MB,), jnp.int32),
            pltpu.VMEM((MB,), jnp.int32),
            pltpu.VMEM((MB + 16,), jnp.int32),
            pltpu.VMEM((MB + 16,), jnp.int32),
            pltpu.VMEM((MB, D), jnp.float32),
            pltpu.VMEM((MB, D), jnp.float32),
            pltpu.VMEM((MB, 128), jnp.float32),
            pltpu.VMEM((MB, 128), jnp.float32),
            pltpu.VMEM((SUB, D), jnp.float32),
            pltpu.VMEM((SUB, 16), jnp.float32),
            pltpu.VMEM((RP_LEN,), jnp.int32),
            pltpu.SemaphoreType.DMA,
            pltpu.SemaphoreType.DMA,
            pltpu.SemaphoreType.DMA,
            pltpu.SemaphoreType.DMA,
            pltpu.SemaphoreType.DMA,
            pltpu.SemaphoreType.DMA,
        ])
    def agg(table_hbm, p_hbm, src_hbm, desc_hbm, rp_hbm, out_hbm,
            idx0, idx1, dsc0, dsc1, rows0, rows1, sc0, sc1,
            stag_v, den_v, rp_v, s1_0, s1_1, sr0, sr1, ss0, ss1):
        idxb = (idx0, idx1)
        dscb = (dsc0, dsc1)
        rowsb = (rows0, rows1)
        scb = (sc0, sc1)
        s1 = (s1_0, s1_1)
        sr = (sr0, sr1)
        ss = (ss0, ss1)
        wid = lax.axis_index("s") * 2 + lax.axis_index("c")
        pltpu.sync_copy(rp_hbm, rp_v)
        zv = jnp.zeros((16,), jnp.float32)

        def sub_body(s, _):
            gsub = wid * subs + s
            base_row = pl.multiple_of(gsub * SUB, 8)
            rp_pair = rp_v[pl.ds(gsub, 16)]
            n_lo = rp_pair[0]
            n_hi = rp_pair[1]
            nb0 = jnp.bitwise_and(n_lo, jnp.int32(~7))
            nblk = (n_hi - nb0 + (MB - 1)) // MB

            def zero_row(e, _):
                for k in range(D // 16):
                    stag_v[e, pl.ds(k * 16, 16)] = zv
                den_v[e, :] = zv
                return 0
            lax.fori_loop(0, SUB, zero_row, 0)

            def issue_s1(j, b):
                nb = pl.multiple_of(nb0 + j * MB, 8)
                pltpu.async_copy(src_hbm.at[pl.ds(nb, MB)], idxb[b], s1[b])
                pltpu.async_copy(desc_hbm.at[pl.ds(nb, MB)],
                                 dscb[b].at[pl.ds(0, MB)], s1[b])

            def wait_s1(b):
                pltpu.make_async_copy(src_hbm.at[pl.ds(0, MB)],
                                      idxb[b], s1[b]).wait()
                pltpu.make_async_copy(desc_hbm.at[pl.ds(0, MB)],
                                      dscb[b].at[pl.ds(0, MB)], s1[b]).wait()

            def issue_s2(b):
                pltpu.async_copy(p_hbm.at[idxb[b]], scb[b], ss[b])
                pltpu.async_copy(table_hbm.at[idxb[b]], rowsb[b], sr[b])

            def wait_s2(b):
                pltpu.make_async_copy(p_hbm.at[idxb[b]], scb[b], ss[b]).wait()
                pltpu.make_async_copy(table_hbm.at[idxb[b]],
                                      rowsb[b], sr[b]).wait()

            def process(j, b):
                nb = nb0 + j * MB
                rows_r = rowsb[b]
                sc_r = scb[b]
                dsc_r = dscb[b]

                @plsc.parallel_loop(0, MB, step=1, unroll=4)
                def member(i):
                    g = nb + i
                    valid = jnp.logical_and(g >= n_lo, g < n_hi)
                    mf = jnp.where(valid, 1.0, 0.0)
                    dsc = dsc_r[pl.ds(i, 16)][0]
                    dloc = lax.shift_right_logical(dsc, 2)
                    c16 = lax.shift_left(jnp.bitwise_and(dsc, 3), 4)
                    srow = sc_r[i, pl.ds(c16, 16)]
                    wrow = jnp.exp(jnp.maximum(srow, srow * NEG_SLOPE)) * mf
                    for h in range(HEADS):
                        ws = wrow[h]
                        for q in range(OUT_CH // 16):
                            c = h * OUT_CH + q * 16
                            plsc.addupdate(stag_v.at[dloc, pl.ds(c, 16)],
                                           rows_r[i, pl.ds(c, 16)] * ws)
                    plsc.addupdate(den_v.at[dloc, :], wrow)

            @pl.when(nblk > 0)
            def _():
                issue_s1(0, 0)

                @pl.when(nblk > 1)
                def _():
                    issue_s1(1, 1)
                wait_s1(0)
                issue_s2(0)

                def pair(jj, _):
                    for b in (0, 1):
                        j = 2 * jj + b

                        @pl.when(j < nblk)
                        def _():
                            wait_s2(b)

                            @pl.when(j + 1 < nblk)
                            def _():
                                wait_s1(1 - b)
                                issue_s2(1 - b)
                            process(j, b)

                            @pl.when(j + 2 < nblk)
                            def _():
                                issue_s1(j + 2, b)
                    return 0
                lax.fori_loop(0, (nblk + 1) // 2, pair, 0)

            def scale_row(e, _):
                rec = 1.0 / (den_v[e, :] + 1e-16)
                for h in range(HEADS):
                    r = rec[h]
                    for q in range(OUT_CH // 16):
                        c = h * OUT_CH + q * 16
                        stag_v[e, pl.ds(c, 16)] = stag_v[e, pl.ds(c, 16)] * r
                return 0
            lax.fori_loop(0, SUB, scale_row, 0)

            pltpu.sync_copy(stag_v, out_hbm.at[pl.ds(base_row, SUB)])
            return 0
        lax.fori_loop(0, subs, sub_body, 0)

    return agg


_agg_a = _make_agg(EDGE_PAD, SUBS_A)
_agg_b = _make_agg(N_PADB, SUBS_B)


def _att_mat(att):
    m = jnp.einsum('chk,hg->hkcg', att, jnp.eye(HEADS, dtype=att.dtype))
    m = jnp.pad(m, ((0, 0), (0, 0), (0, 4), (0, 16 - HEADS)))
    return m.reshape(D, 128)


def _csr(dst_ids, src_ids, cls_ids, n_rows):
    order = jnp.argsort(dst_ids)
    dst_s = dst_ids[order].astype(jnp.int32)
    src_s = src_ids[order].astype(jnp.int32)
    cls_s = cls_ids[order].astype(jnp.int32)
    desc_s = jnp.bitwise_and(dst_s, SUB - 1) * 4 + cls_s
    bases = jnp.arange(0, n_rows + 1, SUB, dtype=jnp.int32)
    rp = jnp.searchsorted(dst_s, bases).astype(jnp.int32)
    rp = jnp.pad(rp, (0, RP_LEN - rp.shape[0]), constant_values=NNZ)
    pad = NNZ_PAD - NNZ
    src_s = jnp.pad(src_s, (0, pad))
    desc_s = jnp.pad(desc_s, (0, pad))
    return src_s, desc_s, rp


def kernel(X, vertex, edges, V_class, E_class, W, b, att_v, att_e):
    Be64 = _att_mat(att_e)
    Bv64 = _att_mat(att_v)
    X0, P = _project(X, W, b.reshape(1, D), Be64)

    srcA, descA, rpA = _csr(edges, vertex, E_class, EDGE_PAD)
    Xe = _agg_a(X0, P, srcA, descA, rpA)

    Q = _score(Xe, Bv64)

    srcB, descB, rpB = _csr(vertex, edges, V_class, N_PADB)
    Xv = _agg_b(Xe, Q, srcB, descB, rpB)
    return Xv[:N]
